# baseline (speedup 1.0000x reference)
import functools

import jax
import jax.numpy as jnp
from jax import lax
from jax.experimental import pallas as pl
from jax.experimental.pallas import tpu as pltpu
from jax.experimental.pallas import tpu_sc as plsc

_SEQ = 200
_HALF = _SEQ // 2
_BATCH = 4096
_EMB = 64
_NCH = _EMB // 16
_NC = 2
_NS = 16
_NW = _NC * _NS
_BPW = _BATCH // _NW


def _sc_segment_sum(xt, table):
    mesh = plsc.VectorSubcoreMesh(core_axis_name="c", subcore_axis_name="s")

    @functools.partial(
        pl.kernel,
        mesh=mesh,
        out_type=jax.ShapeDtypeStruct((_NW, _BPW * _EMB), jnp.float32),
        scratch_types=[
            pltpu.VMEM((2 * _BPW, _HALF), jnp.int32),
            pltpu.VMEM((4, _HALF, _EMB), jnp.float32),
            pltpu.VMEM((_BPW * _EMB,), jnp.float32),
            pltpu.SemaphoreType.DMA,
            pltpu.SemaphoreType.DMA,
            pltpu.SemaphoreType.DMA,
            pltpu.SemaphoreType.DMA,
        ],
        compiler_params=pltpu.CompilerParams(use_tc_tiling_on_sc=False),
    )
    def body(xt_hbm, table_hbm, out_hbm, idx_v, rows_v, res_v,
             s0, s1, s2, s3):
        sems = (s0, s1, s2, s3)
        wid = lax.axis_index("s") * _NC + lax.axis_index("c")
        pltpu.sync_copy(xt_hbm.at[pl.ds(wid * 2 * _BPW, 2 * _BPW), :], idx_v)

        for buf in range(4):
            pltpu.async_copy(
                table_hbm.at[idx_v.at[buf]], rows_v.at[buf], sems[buf])

        def step(i, carry):
            for d in range(2):
                c = 2 * i + d
                acc = [jnp.zeros((16,), jnp.float32) for _ in range(_NCH)]
                for h in range(2):
                    buf = 2 * d + h
                    pltpu.make_async_copy(
                        table_hbm.at[idx_v.at[2 * c + h]],
                        rows_v.at[buf], sems[buf]).wait()
                    for r in range(_HALF):
                        for j in range(_NCH):
                            acc[j] = acc[j] + rows_v[buf, r, pl.ds(16 * j, 16)]

                    @pl.when(c + 2 < _BPW)
                    def _():
                        pltpu.async_copy(
                            table_hbm.at[idx_v.at[2 * c + 4 + h]],
                            rows_v.at[buf], sems[buf])

                for j in range(_NCH):
                    res_v[pl.ds(c * _EMB + 16 * j, 16)] = acc[j]
            return carry

        lax.fori_loop(0, _BPW // 2, step, 0)
        pltpu.sync_copy(res_v, out_hbm.at[wid])

    return body(xt, table)


def _tc_head(sums, W1, b1, W2, b2):
    ncls = W2.shape[0]

    def body(s_ref, w1_ref, b1_ref, w2_ref, b2_ref, o_ref):
        s = s_ref[...] * jnp.float32(1.0 / _SEQ)
        h = lax.dot_general(
            s, w1_ref[...], (((1,), (1,)), ((), ())),
            preferred_element_type=jnp.float32,
        ) + b1_ref[...]
        o = lax.dot_general(
            h, w2_ref[...], (((1,), (1,)), ((), ())),
            preferred_element_type=jnp.float32,
        ) + b2_ref[...]
        o_ref[...] = o

    return pl.pallas_call(
        body,
        out_shape=jax.ShapeDtypeStruct((_BATCH, ncls), jnp.float32),
    )(sums, W1, b1.reshape(1, -1), W2, b2.reshape(1, -1))


def kernel(x, table, W1, b1, W2, b2):
    x = x.astype(jnp.int32)
    xt = x.T.reshape(2 * _BATCH, _HALF)
    sums = _sc_segment_sum(xt, table).reshape(_BATCH, _EMB)
    return _tc_head(sums, W1, b1, W2, b2)


# speedup vs baseline: 1.7144x; 1.7144x over previous
import functools

import jax
import jax.numpy as jnp
from jax import lax
from jax.experimental import pallas as pl
from jax.experimental.pallas import tpu as pltpu
from jax.experimental.pallas import tpu_sc as plsc

_SEQ = 200
_BATCH = 4096
_EMB = 64
_VOCAB = 1000000
_NC = 2
_NS = 16
_NW = _NC * _NS
_BPW = _BATCH // _NW

_BROW = 10240
_NBLK = (_VOCAB + _BROW - 1) // _BROW
_LAST = _VOCAB - (_NBLK - 1) * _BROW
_T2PAD = _NBLK * _BROW
_NBUF = 4


def _project(table, W1, b1, W2, b2):
    def body(tb_hbm, w1_ref, w2_ref, b1_ref, b2_ref,
             a_ref, b_ref, c_ref, bufs, s0, s1, s2, s3):
        sems = (s0, s1, s2, s3)
        mt = lax.dot_general(
            w2_ref[...], w1_ref[...], (((1,), (0,)), ((), ())),
            preferred_element_type=jnp.float32,
        )
        c_ref[...] = lax.dot_general(
            w2_ref[...], b1_ref[...], (((1,), (0,)), ((), ())),
            preferred_element_type=jnp.float32,
        ) + b2_ref[...]

        def nrows(blk):
            return _LAST if blk == _NBLK - 1 else _BROW

        def start(blk):
            n = nrows(blk)
            pltpu.make_async_copy(
                tb_hbm.at[pl.ds(blk * _BROW, n), :],
                bufs.at[blk % _NBUF, pl.ds(0, n), :],
                sems[blk % _NBUF],
            ).start()

        for blk in range(_NBUF):
            start(blk)
        for blk in range(_NBLK):
            n = nrows(blk)
            pltpu.make_async_copy(
                tb_hbm.at[pl.ds(blk * _BROW, n), :],
                bufs.at[blk % _NBUF, pl.ds(0, n), :],
                sems[blk % _NBUF],
            ).wait()
            rt = lax.dot_general(
                mt, bufs[blk % _NBUF], (((1,), (1,)), ((), ())),
                preferred_element_type=jnp.float32,
            )
            a_ref[pl.ds(blk * _BROW, _BROW)] = rt[0, :]
            b_ref[pl.ds(blk * _BROW, _BROW)] = rt[1, :]
            if blk + _NBUF < _NBLK:
                start(blk + _NBUF)

    return pl.pallas_call(
        body,
        in_specs=[
            pl.BlockSpec(memory_space=pl.ANY),
            pl.BlockSpec((128, _EMB), lambda: (0, 0)),
            pl.BlockSpec((2, 128), lambda: (0, 0)),
            pl.BlockSpec((128, 16), lambda: (0, 0)),
            pl.BlockSpec((2, 16), lambda: (0, 0)),
        ],
        out_specs=[
            pl.BlockSpec((_T2PAD,), lambda: (0,)),
            pl.BlockSpec((_T2PAD,), lambda: (0,)),
            pl.BlockSpec((2, 16), lambda: (0, 0)),
        ],
        out_shape=[
            jax.ShapeDtypeStruct((_T2PAD,), jnp.float32),
            jax.ShapeDtypeStruct((_T2PAD,), jnp.float32),
            jax.ShapeDtypeStruct((2, 16), jnp.float32),
        ],
        scratch_shapes=[
            pltpu.VMEM((_NBUF, _BROW, _EMB), jnp.float32),
            pltpu.SemaphoreType.DMA,
            pltpu.SemaphoreType.DMA,
            pltpu.SemaphoreType.DMA,
            pltpu.SemaphoreType.DMA,
        ],
    )(table, W1, W2,
      jnp.broadcast_to(b1.reshape(-1, 1), (128, 16)),
      jnp.broadcast_to(b2.reshape(-1, 1), (2, 16)))


def _sc_pool(x, t2a, t2b, c):
    mesh = plsc.VectorSubcoreMesh(core_axis_name="c", subcore_axis_name="s")

    @functools.partial(
        pl.kernel,
        mesh=mesh,
        out_type=jax.ShapeDtypeStruct((2, _BATCH), jnp.float32),
        scratch_types=[
            pltpu.VMEM((_SEQ, _BPW), jnp.int32),
            pltpu.VMEM((2, _BPW), jnp.float32),
            pltpu.VMEM((2, _BPW), jnp.float32),
            pltpu.VMEM((_BPW,), jnp.float32),
            pltpu.VMEM((_BPW,), jnp.float32),
            pltpu.VMEM((2, 16), jnp.float32),
            pltpu.SemaphoreType.DMA,
            pltpu.SemaphoreType.DMA,
            pltpu.SemaphoreType.DMA,
            pltpu.SemaphoreType.DMA,
        ],
        compiler_params=pltpu.CompilerParams(use_tc_tiling_on_sc=False),
    )
    def body(x_hbm, a_hbm, b_hbm, c_hbm, out_hbm, idx_v, va_v, vb_v,
             ra_v, rb_v, c_v, sa0, sa1, sb0, sb1):
        sas = (sa0, sa1)
        sbs = (sb0, sb1)
        wid = lax.axis_index("s") * _NC + lax.axis_index("c")
        base = wid * _BPW
        pltpu.sync_copy(x_hbm.at[:, pl.ds(base, _BPW)], idx_v)
        pltpu.sync_copy(c_hbm, c_v)

        for d in range(2):
            pltpu.async_copy(a_hbm.at[idx_v.at[d]], va_v.at[d], sas[d])
            pltpu.async_copy(b_hbm.at[idx_v.at[d]], vb_v.at[d], sbs[d])

        nch = _BPW // 16
        zeros = [jnp.zeros((16,), jnp.float32) for _ in range(2 * nch)]

        def step(i, acc):
            acc = list(acc)
            for d in range(2):
                s = 2 * i + d
                pltpu.make_async_copy(
                    a_hbm.at[idx_v.at[s]], va_v.at[d], sas[d]).wait()
                pltpu.make_async_copy(
                    b_hbm.at[idx_v.at[s]], vb_v.at[d], sbs[d]).wait()
                for j in range(nch):
                    acc[j] = acc[j] + va_v[d, pl.ds(16 * j, 16)]
                    acc[nch + j] = acc[nch + j] + vb_v[d, pl.ds(16 * j, 16)]

                @pl.when(s + 2 < _SEQ)
                def _():
                    pltpu.async_copy(
                        a_hbm.at[idx_v.at[s + 2]], va_v.at[d], sas[d])
                    pltpu.async_copy(
                        b_hbm.at[idx_v.at[s + 2]], vb_v.at[d], sbs[d])

            return tuple(acc)

        acc = lax.fori_loop(0, _SEQ // 2, step, tuple(zeros))

        inv = jnp.float32(1.0 / _SEQ)
        ca = c_v[0]
        cb = c_v[1]
        for j in range(nch):
            ra_v[pl.ds(16 * j, 16)] = acc[j] * inv + ca
            rb_v[pl.ds(16 * j, 16)] = acc[nch + j] * inv + cb
        pltpu.sync_copy(ra_v, out_hbm.at[0, pl.ds(base, _BPW)])
        pltpu.sync_copy(rb_v, out_hbm.at[1, pl.ds(base, _BPW)])

    return body(x, t2a, t2b, c)


def kernel(x, table, W1, b1, W2, b2):
    x = x.astype(jnp.int32)
    t2a, t2b, c = _project(table, W1, b1, W2, b2)
    out = _sc_pool(x, t2a, t2b, c)
    return out.T


# speedup vs baseline: 5.1574x; 3.0083x over previous
import functools

import jax
import jax.numpy as jnp
from jax import lax
from jax.experimental import pallas as pl
from jax.experimental.pallas import tpu as pltpu
from jax.experimental.pallas import tpu_sc as plsc

_SEQ = 200
_BATCH = 4096
_EMB = 64
_VOCAB = 1000000
_NC = 2
_NS = 16
_NW = _NC * _NS
_BPW = _BATCH // _NW

_BCOL = 20480
_NBLK = (_VOCAB + _BCOL - 1) // _BCOL
_T2PAD = _NBLK * _BCOL


def _project(tableT, W1, b1, W2, b2):
    def body(tb_ref, w1_ref, w2_ref, b1_ref, b2_ref, a_ref, b_ref, c_ref):
        mt = lax.dot_general(
            w2_ref[...], w1_ref[...], (((1,), (0,)), ((), ())),
            preferred_element_type=jnp.float32,
        )
        rt = lax.dot_general(
            mt, tb_ref[...], (((1,), (0,)), ((), ())),
            preferred_element_type=jnp.float32,
        )
        a_ref[...] = rt[0, :]
        b_ref[...] = rt[1, :]
        c_ref[...] = lax.dot_general(
            w2_ref[...], b1_ref[...], (((1,), (0,)), ((), ())),
            preferred_element_type=jnp.float32,
        ) + b2_ref[...]

    return pl.pallas_call(
        body,
        grid=(_NBLK,),
        in_specs=[
            pl.BlockSpec((_EMB, _BCOL), lambda i: (0, i)),
            pl.BlockSpec((128, _EMB), lambda i: (0, 0)),
            pl.BlockSpec((2, 128), lambda i: (0, 0)),
            pl.BlockSpec((128, 16), lambda i: (0, 0)),
            pl.BlockSpec((2, 16), lambda i: (0, 0)),
        ],
        out_specs=[
            pl.BlockSpec((_BCOL,), lambda i: (i,)),
            pl.BlockSpec((_BCOL,), lambda i: (i,)),
            pl.BlockSpec((2, 16), lambda i: (0, 0)),
        ],
        out_shape=[
            jax.ShapeDtypeStruct((_T2PAD,), jnp.float32),
            jax.ShapeDtypeStruct((_T2PAD,), jnp.float32),
            jax.ShapeDtypeStruct((2, 16), jnp.float32),
        ],
    )(tableT, W1, W2,
      jnp.broadcast_to(b1.reshape(-1, 1), (128, 16)),
      jnp.broadcast_to(b2.reshape(-1, 1), (2, 16)))


def _sc_pool(x, t2a, t2b, c):
    mesh = plsc.VectorSubcoreMesh(core_axis_name="c", subcore_axis_name="s")

    @functools.partial(
        pl.kernel,
        mesh=mesh,
        out_type=jax.ShapeDtypeStruct((2, _BATCH), jnp.float32),
        scratch_types=[
            pltpu.VMEM((_SEQ, _BPW), jnp.int32),
            pltpu.VMEM((2, _BPW), jnp.float32),
            pltpu.VMEM((2, _BPW), jnp.float32),
            pltpu.VMEM((_BPW,), jnp.float32),
            pltpu.VMEM((_BPW,), jnp.float32),
            pltpu.VMEM((2, 16), jnp.float32),
            pltpu.SemaphoreType.DMA,
            pltpu.SemaphoreType.DMA,
            pltpu.SemaphoreType.DMA,
            pltpu.SemaphoreType.DMA,
        ],
        compiler_params=pltpu.CompilerParams(use_tc_tiling_on_sc=False),
    )
    def body(x_hbm, a_hbm, b_hbm, c_hbm, out_hbm, idx_v, va_v, vb_v,
             ra_v, rb_v, c_v, sa0, sa1, sb0, sb1):
        sas = (sa0, sa1)
        sbs = (sb0, sb1)
        wid = lax.axis_index("s") * _NC + lax.axis_index("c")
        base = wid * _BPW
        pltpu.sync_copy(x_hbm.at[:, pl.ds(base, _BPW)], idx_v)
        pltpu.sync_copy(c_hbm, c_v)

        for d in range(2):
            pltpu.async_copy(a_hbm.at[idx_v.at[d]], va_v.at[d], sas[d])
            pltpu.async_copy(b_hbm.at[idx_v.at[d]], vb_v.at[d], sbs[d])

        nch = _BPW // 16
        zeros = [jnp.zeros((16,), jnp.float32) for _ in range(2 * nch)]

        def step(i, acc):
            acc = list(acc)
            for d in range(2):
                s = 2 * i + d
                pltpu.make_async_copy(
                    a_hbm.at[idx_v.at[s]], va_v.at[d], sas[d]).wait()
                pltpu.make_async_copy(
                    b_hbm.at[idx_v.at[s]], vb_v.at[d], sbs[d]).wait()
                for j in range(nch):
                    acc[j] = acc[j] + va_v[d, pl.ds(16 * j, 16)]
                    acc[nch + j] = acc[nch + j] + vb_v[d, pl.ds(16 * j, 16)]

                @pl.when(s + 2 < _SEQ)
                def _():
                    pltpu.async_copy(
                        a_hbm.at[idx_v.at[s + 2]], va_v.at[d], sas[d])
                    pltpu.async_copy(
                        b_hbm.at[idx_v.at[s + 2]], vb_v.at[d], sbs[d])

            return tuple(acc)

        acc = lax.fori_loop(0, _SEQ // 2, step, tuple(zeros))

        inv = jnp.float32(1.0 / _SEQ)
        ca = c_v[0]
        cb = c_v[1]
        for j in range(nch):
            ra_v[pl.ds(16 * j, 16)] = acc[j] * inv + ca
            rb_v[pl.ds(16 * j, 16)] = acc[nch + j] * inv + cb
        pltpu.sync_copy(ra_v, out_hbm.at[0, pl.ds(base, _BPW)])
        pltpu.sync_copy(rb_v, out_hbm.at[1, pl.ds(base, _BPW)])

    return body(x, t2a, t2b, c)


def kernel(x, table, W1, b1, W2, b2):
    x = x.astype(jnp.int32)
    t2a, t2b, c = _project(table.T, W1, b1, W2, b2)
    out = _sc_pool(x, t2a, t2b, c)
    return out.T


# speedup vs baseline: 5.4374x; 1.0543x over previous
import functools

import jax
import jax.numpy as jnp
from jax import lax
from jax.experimental import pallas as pl
from jax.experimental.pallas import tpu as pltpu
from jax.experimental.pallas import tpu_sc as plsc

_SEQ = 200
_BATCH = 4096
_EMB = 64
_VOCAB = 1000000
_NC = 2
_NS = 16
_NW = _NC * _NS
_BPW = _BATCH // _NW

_BCOL = 20480
_NBLK = (_VOCAB + _BCOL - 1) // _BCOL
_T2PAD = _NBLK * _BCOL


def _project(tableT, W1, b1, W2, b2):
    def body(tb_ref, w1_ref, w2_ref, b1_ref, b2_ref, p_ref, c_ref):
        mt = lax.dot_general(
            w2_ref[...], w1_ref[...], (((1,), (0,)), ((), ())),
            preferred_element_type=jnp.float32,
        )
        rt = lax.dot_general(
            mt, tb_ref[...], (((1,), (0,)), ((), ())),
            preferred_element_type=jnp.float32,
        )

        def rne16(v):
            u = lax.bitcast_convert_type(v, jnp.uint32)
            return (u + jnp.uint32(0x7FFF) + ((u >> 16) & jnp.uint32(1))) >> 16

        packed = (rne16(rt[0, :]) << 16) | rne16(rt[1, :])
        p_ref[...] = lax.bitcast_convert_type(packed, jnp.float32)
        c_ref[...] = lax.dot_general(
            w2_ref[...], b1_ref[...], (((1,), (0,)), ((), ())),
            preferred_element_type=jnp.float32,
        ) + b2_ref[...]

    return pl.pallas_call(
        body,
        grid=(_NBLK,),
        in_specs=[
            pl.BlockSpec((_EMB, _BCOL), lambda i: (0, i)),
            pl.BlockSpec((128, _EMB), lambda i: (0, 0)),
            pl.BlockSpec((2, 128), lambda i: (0, 0)),
            pl.BlockSpec((128, 16), lambda i: (0, 0)),
            pl.BlockSpec((2, 16), lambda i: (0, 0)),
        ],
        out_specs=[
            pl.BlockSpec((_BCOL,), lambda i: (i,)),
            pl.BlockSpec((2, 16), lambda i: (0, 0)),
        ],
        out_shape=[
            jax.ShapeDtypeStruct((_T2PAD,), jnp.float32),
            jax.ShapeDtypeStruct((2, 16), jnp.float32),
        ],
    )(tableT, W1, W2,
      jnp.broadcast_to(b1.reshape(-1, 1), (128, 16)),
      jnp.broadcast_to(b2.reshape(-1, 1), (2, 16)))


def _sc_pool(x, t2p, c):
    mesh = plsc.VectorSubcoreMesh(core_axis_name="c", subcore_axis_name="s")
    hi_mask = jnp.uint32(0xFFFF0000)

    @functools.partial(
        pl.kernel,
        mesh=mesh,
        out_type=jax.ShapeDtypeStruct((2, _BATCH), jnp.float32),
        scratch_types=[
            pltpu.VMEM((_SEQ, _BPW), jnp.int32),
            pltpu.VMEM((2, _BPW), jnp.float32),
            pltpu.VMEM((_BPW,), jnp.float32),
            pltpu.VMEM((_BPW,), jnp.float32),
            pltpu.VMEM((2, 16), jnp.float32),
            pltpu.SemaphoreType.DMA,
            pltpu.SemaphoreType.DMA,
        ],
        compiler_params=pltpu.CompilerParams(
            use_tc_tiling_on_sc=False, needs_layout_passes=False),
    )
    def body(x_hbm, p_hbm, c_hbm, out_hbm, idx_v, vp_v,
             ra_v, rb_v, c_v, s0, s1):
        sems = (s0, s1)
        wid = lax.axis_index("s") * _NC + lax.axis_index("c")
        base = wid * _BPW
        pltpu.sync_copy(x_hbm.at[:, pl.ds(base, _BPW)], idx_v)
        pltpu.sync_copy(c_hbm, c_v)

        for d in range(2):
            pltpu.async_copy(p_hbm.at[idx_v.at[d]], vp_v.at[d], sems[d])

        nch = _BPW // 16
        zeros = [jnp.zeros((16,), jnp.float32) for _ in range(2 * nch)]

        def step(i, acc):
            acc = list(acc)
            for d in range(2):
                s = 2 * i + d
                pltpu.make_async_copy(
                    p_hbm.at[idx_v.at[s]], vp_v.at[d], sems[d]).wait()
                for j in range(nch):
                    u = plsc.bitcast(vp_v[d, pl.ds(16 * j, 16)], jnp.uint32)
                    av = plsc.bitcast(u & hi_mask, jnp.float32)
                    bv = plsc.bitcast(u << 16, jnp.float32)
                    acc[j] = acc[j] + av
                    acc[nch + j] = acc[nch + j] + bv

                @pl.when(s + 2 < _SEQ)
                def _():
                    pltpu.async_copy(
                        p_hbm.at[idx_v.at[s + 2]], vp_v.at[d], sems[d])

            return tuple(acc)

        acc = lax.fori_loop(0, _SEQ // 2, step, tuple(zeros))

        inv = jnp.float32(1.0 / _SEQ)
        ca = c_v[0]
        cb = c_v[1]
        for j in range(nch):
            ra_v[pl.ds(16 * j, 16)] = acc[j] * inv + ca
            rb_v[pl.ds(16 * j, 16)] = acc[nch + j] * inv + cb
        pltpu.sync_copy(ra_v, out_hbm.at[0, pl.ds(base, _BPW)])
        pltpu.sync_copy(rb_v, out_hbm.at[1, pl.ds(base, _BPW)])

    return body(x, t2p, c)


def kernel(x, table, W1, b1, W2, b2):
    x = x.astype(jnp.int32)
    t2p, c = _project(table.T, W1, b1, W2, b2)
    out = _sc_pool(x, t2p, c)
    return out.T
